# baseline (speedup 1.0000x reference)
import functools

import jax
import jax.numpy as jnp
from jax import lax
from jax.experimental import pallas as pl
from jax.experimental.pallas import tpu as pltpu
from jax.experimental.pallas import tpu_sc as plsc

_LANES = 16


@functools.cache
def _sc_call(BS, B, L, D):
  info = plsc.get_sparse_core_info()
  NC, NS = info.num_cores, info.num_subcores
  NW = NC * NS
  LW = L // NW
  C = min(LW, 8)
  NCHUNK = LW // C
  NV = C * D // _LANES
  NCOL = D // _LANES
  mesh = plsc.VectorSubcoreMesh(core_axis_name="c", subcore_axis_name="s")

  @functools.partial(
      pl.kernel,
      out_type=jax.ShapeDtypeStruct((B, L, D), jnp.float32),
      mesh=mesh,
      scratch_types=[
          [pltpu.VMEM((C, D), jnp.float32)] * 2,
          [[pltpu.VMEM((C, D), jnp.float32)] * 3 for _ in range(BS)],
          [pltpu.SemaphoreType.DMA] * 2,
          [[pltpu.SemaphoreType.DMA] * 3 for _ in range(BS)],
          [[pltpu.SemaphoreType.DMA] * 3 for _ in range(BS)],
      ],
  )
  def k(x_hbm, pos_hbm, out_hbm, pos_v, xb_v, sp, sl, ss):
    wid = lax.axis_index("s") * NC + lax.axis_index("c")
    lbase = wid * LW

    def row(lc):
      return pl.multiple_of(lbase + lc * C, C)

    def pos_load(lc):
      return pltpu.async_copy(pos_hbm.at[pl.ds(row(lc), C)], pos_v[lc % 2],
                              sp[lc % 2])

    def x_load(lc, b):
      return pltpu.async_copy(x_hbm.at[b, pl.ds(row(lc), C)],
                              xb_v[b][lc % 3], sl[b][lc % 3])

    def x_store(lc, b):
      return pltpu.async_copy(xb_v[b][lc % 3],
                              out_hbm.at[b, pl.ds(row(lc), C)],
                              ss[b][lc % 3])

    pos_d = [None] * NCHUNK
    loads = [[None] * BS for _ in range(NCHUNK)]
    stores = [[None] * BS for _ in range(NCHUNK)]
    pos_d[0] = pos_load(0)
    for b in range(BS):
      loads[0][b] = x_load(0, b)
    if NCHUNK > 1:
      pos_d[1] = pos_load(1)
      for b in range(BS):
        loads[1][b] = x_load(1, b)

    for lc in range(NCHUNK):
      pos_d[lc].wait()
      for b in range(BS):
        loads[lc][b].wait()
      bufs = [xb_v[b][lc % 3] for b in range(BS)]
      pos = pos_v[lc % 2]

      @plsc.parallel_loop(0, NV, unroll=4)
      def _(i, _bufs=bufs, _pos=pos):
        r = i // NCOL
        s = pl.ds((i % NCOL) * _LANES, _LANES)
        p = _pos[r, s]
        for _b in _bufs:
          _b[r, s] = _b[r, s] + p

      for b in range(BS):
        stores[lc][b] = x_store(lc, b)
      if lc + 2 < NCHUNK:
        pos_d[lc + 2] = pos_load(lc + 2)
        for b in range(BS):
          if lc >= 1:
            stores[lc - 1][b].wait()
          loads[lc + 2][b] = x_load(lc + 2, b)

    for lc in range(max(0, NCHUNK - 3), NCHUNK):
      for b in range(BS):
        if stores[lc][b] is not None:
          stores[lc][b].wait()

  return k


@functools.cache
def _tc_call(BS, B, L, D, BL=256):
  def body(x_ref, pos_ref, sc_ref, o_ref):
    del sc_ref
    o_ref[...] = x_ref[...] + pos_ref[...][None]

  return pl.pallas_call(
      body,
      grid=(L // BL, B - BS),
      in_specs=[
          pl.BlockSpec((1, BL, D), lambda l, t: (t + BS, l, 0)),
          pl.BlockSpec((BL, D), lambda l, t: (l, 0)),
          pl.BlockSpec(memory_space=pl.ANY),
      ],
      out_specs=pl.BlockSpec((1, BL, D), lambda l, t: (t + BS, l, 0)),
      out_shape=jax.ShapeDtypeStruct((B, L, D), jnp.float32),
      input_output_aliases={2: 0},
  )


def kernel(x, pos_table):
  B, L, D = x.shape
  BS = 2
  pos = pos_table[:L]
  sc_out = _sc_call(BS, B, L, D)(x, pos)
  return _tc_call(BS, B, L, D)(x, pos, sc_out)


# speedup vs baseline: 1.1192x; 1.1192x over previous
import functools

import jax
import jax.numpy as jnp
from jax import lax
from jax.experimental import pallas as pl
from jax.experimental.pallas import tpu as pltpu
from jax.experimental.pallas import tpu_sc as plsc

_LANES = 16


@functools.cache
def _sc_call(BS, B, L, D):
  info = plsc.get_sparse_core_info()
  NC, NS = info.num_cores, info.num_subcores
  NW = NC * NS
  LW = L // NW
  C = min(LW, 8)
  NCHUNK = LW // C
  NV = C * D // _LANES
  NCOL = D // _LANES
  mesh = plsc.VectorSubcoreMesh(core_axis_name="c", subcore_axis_name="s")

  @functools.partial(
      pl.kernel,
      out_type=jax.ShapeDtypeStruct((B, L, D), jnp.float32),
      mesh=mesh,
      scratch_types=[
          [pltpu.VMEM((C, D), jnp.float32)] * 2,
          [[pltpu.VMEM((C, D), jnp.float32)] * 3 for _ in range(BS)],
          [pltpu.SemaphoreType.DMA] * 2,
          [[pltpu.SemaphoreType.DMA] * 3 for _ in range(BS)],
          [[pltpu.SemaphoreType.DMA] * 3 for _ in range(BS)],
      ],
  )
  def k(x_hbm, pos_hbm, out_hbm, pos_v, xb_v, sp, sl, ss):
    wid = lax.axis_index("s") * NC + lax.axis_index("c")
    lbase = wid * LW

    def row(lc):
      return pl.multiple_of(lbase + lc * C, C)

    def pos_load(lc):
      return pltpu.async_copy(pos_hbm.at[pl.ds(row(lc), C)], pos_v[lc % 2],
                              sp[lc % 2])

    def x_load(lc, b):
      return pltpu.async_copy(x_hbm.at[b, pl.ds(row(lc), C)],
                              xb_v[b][lc % 3], sl[b][lc % 3])

    def x_store(lc, b):
      return pltpu.async_copy(xb_v[b][lc % 3],
                              out_hbm.at[b, pl.ds(row(lc), C)],
                              ss[b][lc % 3])

    pos_d = [None] * NCHUNK
    loads = [[None] * BS for _ in range(NCHUNK)]
    stores = [[None] * BS for _ in range(NCHUNK)]
    pos_d[0] = pos_load(0)
    for b in range(BS):
      loads[0][b] = x_load(0, b)
    if NCHUNK > 1:
      pos_d[1] = pos_load(1)
      for b in range(BS):
        loads[1][b] = x_load(1, b)

    for lc in range(NCHUNK):
      pos_d[lc].wait()
      for b in range(BS):
        loads[lc][b].wait()
      bufs = [xb_v[b][lc % 3] for b in range(BS)]
      pos = pos_v[lc % 2]

      @plsc.parallel_loop(0, NV, unroll=4)
      def _(i, _bufs=bufs, _pos=pos):
        r = i // NCOL
        s = pl.ds((i % NCOL) * _LANES, _LANES)
        p = _pos[r, s]
        for _b in _bufs:
          _b[r, s] = _b[r, s] + p

      for b in range(BS):
        stores[lc][b] = x_store(lc, b)
      if lc + 2 < NCHUNK:
        pos_d[lc + 2] = pos_load(lc + 2)
        for b in range(BS):
          if lc >= 1:
            stores[lc - 1][b].wait()
          loads[lc + 2][b] = x_load(lc + 2, b)

    for lc in range(max(0, NCHUNK - 3), NCHUNK):
      for b in range(BS):
        if stores[lc][b] is not None:
          stores[lc][b].wait()

  return k


@functools.cache
def _tc_call(BS, B, L, D, BL=1024):
  def body(x_ref, pos_ref, sc_ref, o_ref):
    del sc_ref
    o_ref[...] = x_ref[...] + pos_ref[...][None]

  return pl.pallas_call(
      body,
      grid=(L // BL, B - BS),
      in_specs=[
          pl.BlockSpec((1, BL, D), lambda l, t: (t + BS, l, 0)),
          pl.BlockSpec((BL, D), lambda l, t: (l, 0)),
          pl.BlockSpec(memory_space=pl.ANY),
      ],
      out_specs=pl.BlockSpec((1, BL, D), lambda l, t: (t + BS, l, 0)),
      out_shape=jax.ShapeDtypeStruct((B, L, D), jnp.float32),
      input_output_aliases={2: 0},
  )


def kernel(x, pos_table):
  B, L, D = x.shape
  BS = 2
  pos = pos_table[:L]
  sc_out = _sc_call(BS, B, L, D)(x, pos)
  return _tc_call(BS, B, L, D)(x, pos, sc_out)
